# baseline (speedup 1.0000x reference)
import functools

import jax
import jax.numpy as jnp
from jax import lax
from jax.experimental import pallas as pl
from jax.experimental.pallas import tpu as pltpu
from jax.experimental.pallas import tpu_sc as plsc

_N = 50000
_D = 128
_S = 6
_E = 30000

_NP = 50176
_EP = 30720
_NREL = 2 * _S + 2
_ET = _NREL * _EP
_NW = 32
_CHUNK = 128
_TM = 512
_CW = 8
_NQ = _D // _CW
_NPT = _NP // 16
_DUMP = _N
_EPS = 1e-5


def _ln(x, g, b):
    mu = jnp.mean(x, axis=-1, keepdims=True)
    var = jnp.mean((x - mu) ** 2, axis=-1, keepdims=True)
    return (x - mu) * jax.lax.rsqrt(var + _EPS) * g + b


def _mlp_body(xc_ref, xs_ref, w1_ref, b1_ref, w2_ref, gg_ref, gb_ref,
              sw1_ref, sb1_ref, sw2_ref, sgg_ref, sgb_ref, o_ref):
    f32 = jnp.float32
    h1 = jnp.maximum(
        lax.dot_general(xc_ref[...], w1_ref[...], (((1,), (0,)), ((), ())),
                        preferred_element_type=f32) + b1_ref[...], 0.0)
    h = _ln(lax.dot_general(h1, w2_ref[...], (((1,), (0,)), ((), ())),
                            preferred_element_type=f32), gg_ref[...], gb_ref[...])
    s1 = jnp.maximum(
        lax.dot_general(xs_ref[...], sw1_ref[...], (((1,), (0,)), ((), ())),
                        preferred_element_type=f32) + sb1_ref[...], 0.0)
    hs = _ln(lax.dot_general(s1, sw2_ref[...], (((1,), (0,)), ((), ())),
                             preferred_element_type=f32), sgg_ref[...], sgb_ref[...])
    o_ref[...] = jnp.maximum(h + hs, 0.0)


def _mlp_call(xc, xs, w1p, b1, w2, gg, gb, sw1p, sb1, sw2, sgg, sgb):
    row = pl.BlockSpec((_TM, _D), lambda i: (i, 0))
    full = pl.BlockSpec((_D, _D), lambda i: (0, 0))
    vec = pl.BlockSpec((1, _D), lambda i: (0, 0))
    return pl.pallas_call(
        _mlp_body,
        grid=(_NP // _TM,),
        in_specs=[row, row, full, vec, full, vec, vec, full, vec, full, vec, vec],
        out_specs=row,
        out_shape=jax.ShapeDtypeStruct((_NP, _D), jnp.float32),
    )(xc, xs, w1p, b1, w2, gg, gb, sw1p, sb1, sw2, sgg, sgb)


_sc_mesh = plsc.VectorSubcoreMesh(core_axis_name="c", subcore_axis_name="s")
_sc_params = pltpu.CompilerParams(use_tc_tiling_on_sc=False)


@functools.partial(
    pl.kernel,
    mesh=_sc_mesh,
    compiler_params=_sc_params,
    out_type=jax.ShapeDtypeStruct((_ET, _D), jnp.float32),
    scratch_types=[
        pltpu.VMEM((_CHUNK,), jnp.int32),
        pltpu.VMEM((_CHUNK, _D), jnp.float32),
        pltpu.SemaphoreType.DMA,
    ],
)
def _gather_rows(feat_hbm, src_hbm, out_hbm, idx_v, rows_v, sem):
    wid = lax.axis_index("s") * 2 + lax.axis_index("c")
    per_w = _ET // _NW
    base = wid * per_w

    def body(ci, _):
        off = base + ci * _CHUNK
        pltpu.sync_copy(src_hbm.at[pl.ds(off, _CHUNK)], idx_v)
        pltpu.async_copy(feat_hbm.at[idx_v], rows_v, sem).wait()
        pltpu.sync_copy(rows_v, out_hbm.at[pl.ds(off, _CHUNK)])
        return ()

    lax.fori_loop(0, per_w // _CHUNK, body, ())


def _relmm_body(x_ref, w_ref, o_ref):
    o_ref[0] = lax.dot_general(x_ref[...], w_ref[0], (((1,), (0,)), ((), ())),
                               preferred_element_type=jnp.float32)


def _relmm_call(g, w_split):
    rows_per_rel = _EP // _TM
    return pl.pallas_call(
        _relmm_body,
        grid=(_ET // _TM, _NQ),
        in_specs=[
            pl.BlockSpec((_TM, _D), lambda i, q: (i, 0)),
            pl.BlockSpec((1, _D, _CW),
                         lambda i, q, r=rows_per_rel: ((i // r) * _NQ + q, 0, 0)),
        ],
        out_specs=pl.BlockSpec((1, _TM, _CW), lambda i, q: (q, i, 0)),
        out_shape=jax.ShapeDtypeStruct((_NQ, _ET, _CW), jnp.float32),
    )(g, w_split)


@functools.partial(
    pl.kernel,
    mesh=_sc_mesh,
    compiler_params=_sc_params,
    out_type=jax.ShapeDtypeStruct((_NQ, _NP, _CW), jnp.float32),
    scratch_types=[
        pltpu.VMEM_SHARED((_NP, _CW), jnp.float32),
        pltpu.VMEM((_CHUNK,), jnp.int32),
        pltpu.VMEM((_CHUNK, _CW), jnp.float32),
        pltpu.VMEM((_NPT, _CW), jnp.float32),
        pltpu.SemaphoreType.DMA,
    ],
)
def _scatter_add(p4_hbm, dst_hbm, z_hbm, out_hbm, acc_sh, idx_v, pay_v, buf_v,
                 sem):
    c = lax.axis_index("c")
    s = lax.axis_index("s")
    ept = _ET // 16
    nbase = s * _NPT

    for r in range(_NQ // 2):
        q = 2 * r + c
        pltpu.sync_copy(z_hbm, buf_v)
        pltpu.sync_copy(buf_v, acc_sh.at[pl.ds(nbase, _NPT), :])
        plsc.subcore_barrier()

        def body(ci, _):
            off = s * ept + ci * _CHUNK
            pltpu.sync_copy(dst_hbm.at[pl.ds(off, _CHUNK)], idx_v)
            pltpu.sync_copy(p4_hbm.at[q, pl.ds(off, _CHUNK), :], pay_v)
            pltpu.sync_copy(pay_v, acc_sh.at[idx_v], add=True)
            return ()

        lax.fori_loop(0, ept // _CHUNK, body, ())
        plsc.subcore_barrier()
        pltpu.sync_copy(acc_sh.at[pl.ds(nbase, _NPT), :], buf_v)
        pltpu.sync_copy(buf_v, out_hbm.at[q, pl.ds(nbase, _NPT), :])
        plsc.subcore_barrier()


def _post_body(f_ref, a_ref, cw_ref, c2w_ref, g1_ref, b1_ref, g2_ref, b2_ref,
               o_ref):
    f32 = jnp.float32
    acc = jnp.concatenate([a_ref[j] for j in range(_NQ)], axis=-1)
    temp = lax.dot_general(f_ref[...], cw_ref[...], (((1,), (0,)), ((), ())),
                           preferred_element_type=f32) + acc
    u = jnp.maximum(_ln(temp, g1_ref[...], b1_ref[...]), 0.0)
    v = _ln(lax.dot_general(u, c2w_ref[...], (((1,), (0,)), ((), ())),
                            preferred_element_type=f32), g2_ref[...], b2_ref[...])
    o_ref[...] = jnp.maximum(v + f_ref[...], 0.0)


def _post_call(feat, acc4, cw, c2w, g1, b1, g2, b2):
    row = pl.BlockSpec((_TM, _D), lambda i: (i, 0))
    full = pl.BlockSpec((_D, _D), lambda i: (0, 0))
    vec = pl.BlockSpec((1, _D), lambda i: (0, 0))
    return pl.pallas_call(
        _post_body,
        grid=(_NP // _TM,),
        in_specs=[
            row,
            pl.BlockSpec((_NQ, _TM, _CW), lambda i: (0, i, 0)),
            full, full, vec, vec, vec, vec,
        ],
        out_specs=row,
        out_shape=jax.ShapeDtypeStruct((_NP, _D), jnp.float32),
    )(feat, acc4, cw, c2w, g1, b1, g2, b2)


def _prep_indices(pre, suc, left, right):
    srcs, dsts = [], []
    for k2 in range(_S):
        dsts.append(pre[2 * k2])
        srcs.append(pre[2 * k2 + 1])
    for k2 in range(_S):
        dsts.append(suc[2 * k2])
        srcs.append(suc[2 * k2 + 1])
    dsts.append(left[0])
    srcs.append(left[1])
    dsts.append(right[0])
    srcs.append(right[1])
    pad = _EP - _E
    src_all = jnp.concatenate([jnp.pad(x, (0, pad)) for x in srcs])
    dst_all = jnp.concatenate(
        [jnp.pad(x, (0, pad), constant_values=_DUMP) for x in dsts])
    return src_all, dst_all


def kernel(idcs, ctrs, feats, turn, control, intersect, pre, suc, left, right,
           in_w1, in_b1, in_w2, in_gn_g, in_gn_b, seg_w1, seg_b1, seg_w2,
           seg_gn_g, seg_gn_b, ctr_w, pre_w, suc_w, left_w, right_w, norm_g,
           norm_b, ctr2_w, ctr2_gn_g, ctr2_gn_b):
    f32 = jnp.float32
    padn = _NP - _N
    xc = jnp.pad(ctrs.reshape(_N, 2), ((0, padn), (0, _D - 2)))
    xs = jnp.pad(feats[0], ((0, padn), (0, _D - 2)))
    w1p = jnp.pad(in_w1, ((0, _D - 2), (0, 0)))
    sw1p = jnp.pad(seg_w1, ((0, _D - 2), (0, 0)))
    r1 = lambda v: v.reshape(1, _D)

    feat = _mlp_call(xc, xs, w1p, r1(in_b1), in_w2, r1(in_gn_g), r1(in_gn_b),
                     sw1p, r1(seg_b1), seg_w2, r1(seg_gn_g), r1(seg_gn_b))

    src_all, dst_all = _prep_indices(pre, suc, left, right)
    zblk = jnp.zeros((_NPT, _CW), f32)

    for i in range(4):
        w_rel4 = jnp.concatenate(
            [pre_w[i], suc_w[i], left_w[i][None], right_w[i][None]], axis=0)
        w_split = w_rel4.reshape(_NREL, _D, _NQ, _CW).transpose(
            0, 2, 1, 3).reshape(_NREL * _NQ, _D, _CW)
        g = _gather_rows(feat, src_all)
        p4 = _relmm_call(g, w_split)
        acc4 = _scatter_add(p4, dst_all, zblk)
        feat = _post_call(feat, acc4, ctr_w[i], ctr2_w[i], r1(norm_g[i]),
                          r1(norm_b[i]), r1(ctr2_gn_g[i]), r1(ctr2_gn_b[i]))

    return feat[:_N], idcs, ctrs


# speedup vs baseline: 6.2220x; 6.2220x over previous
import functools

import jax
import jax.numpy as jnp
from jax import lax
from jax.experimental import pallas as pl
from jax.experimental.pallas import tpu as pltpu
from jax.experimental.pallas import tpu_sc as plsc

_N = 50000
_D = 128
_S = 6
_E = 30000

_NP = 50176
_EP = 30720
_NREL = 2 * _S + 2
_ET = _NREL * _EP
_NW = 32
_CHUNK = 128
_KB = 5
_TM = 512
_CW = 16
_NQ = _D // _CW
_NPT = _NP // 16
_DUMP = _N
_EPS = 1e-5


def _ln(x, g, b):
    mu = jnp.mean(x, axis=-1, keepdims=True)
    var = jnp.mean((x - mu) ** 2, axis=-1, keepdims=True)
    return (x - mu) * jax.lax.rsqrt(var + _EPS) * g + b


def _mm(a, b):
    return lax.dot_general(a, b, (((1,), (0,)), ((), ())),
                           preferred_element_type=jnp.float32)


def _mlp_body(xc_ref, xs_ref, w1_ref, b1_ref, w2_ref, gg_ref, gb_ref,
              sw1_ref, sb1_ref, sw2_ref, sgg_ref, sgb_ref, o_ref):
    h1 = jnp.maximum(_mm(xc_ref[...], w1_ref[...]) + b1_ref[...], 0.0)
    h = _ln(_mm(h1, w2_ref[...]), gg_ref[...], gb_ref[...])
    s1 = jnp.maximum(_mm(xs_ref[...], sw1_ref[...]) + sb1_ref[...], 0.0)
    hs = _ln(_mm(s1, sw2_ref[...]), sgg_ref[...], sgb_ref[...])
    o_ref[...] = jnp.maximum(h + hs, 0.0)


def _mlp_call(xc, xs, w1p, b1, w2, gg, gb, sw1p, sb1, sw2, sgg, sgb):
    row = pl.BlockSpec((_TM, _D), lambda i: (i, 0))
    full = pl.BlockSpec((_D, _D), lambda i: (0, 0))
    vec = pl.BlockSpec((1, _D), lambda i: (0, 0))
    return pl.pallas_call(
        _mlp_body,
        grid=(_NP // _TM,),
        in_specs=[row, row, full, vec, full, vec, vec, full, vec, full, vec, vec],
        out_specs=row,
        out_shape=jax.ShapeDtypeStruct((_NP, _D), jnp.float32),
    )(xc, xs, w1p, b1, w2, gg, gb, sw1p, sb1, sw2, sgg, sgb)


_sc_mesh = plsc.VectorSubcoreMesh(core_axis_name="c", subcore_axis_name="s")
_sc_params = pltpu.CompilerParams(use_tc_tiling_on_sc=False)


@functools.partial(
    pl.kernel,
    mesh=_sc_mesh,
    compiler_params=_sc_params,
    out_type=jax.ShapeDtypeStruct((_ET, _D), jnp.float32),
    scratch_types=[
        pltpu.VMEM((_KB, _CHUNK), jnp.int32),
        pltpu.VMEM((_KB, _CHUNK, _D), jnp.float32),
        pltpu.SemaphoreType.DMA,
        pltpu.SemaphoreType.DMA,
    ],
)
def _gather_rows(feat_hbm, src_hbm, out_hbm, idx_v, rows_v, gsem, ssem):
    wid = lax.axis_index("s") * 2 + lax.axis_index("c")
    per_w = _ET // _NW
    base = wid * per_w
    nsup = per_w // (_KB * _CHUNK)

    def body(si, _):
        off0 = base + si * (_KB * _CHUNK)
        for j in range(_KB):
            pltpu.sync_copy(src_hbm.at[pl.ds(off0 + j * _CHUNK, _CHUNK)],
                            idx_v.at[j])
        ghs = [pltpu.async_copy(feat_hbm.at[idx_v.at[j]], rows_v.at[j], gsem)
               for j in range(_KB)]
        for h in ghs:
            h.wait()
        shs = [pltpu.async_copy(
                   rows_v.at[j],
                   out_hbm.at[pl.ds(off0 + j * _CHUNK, _CHUNK)], ssem)
               for j in range(_KB)]
        for h in shs:
            h.wait()
        return ()

    lax.fori_loop(0, nsup, body, ())


def _relmm_body(x_ref, w_ref, o_ref):
    o_ref[...] = _mm(x_ref[...], w_ref[0])


def _relmm_call(g, w_rel):
    rows_per_rel = _EP // _TM
    return pl.pallas_call(
        _relmm_body,
        grid=(_ET // _TM,),
        in_specs=[
            pl.BlockSpec((_TM, _D), lambda i: (i, 0)),
            pl.BlockSpec((1, _D, _D), lambda i, r=rows_per_rel: (i // r, 0, 0)),
        ],
        out_specs=pl.BlockSpec((_TM, _D), lambda i: (i, 0)),
        out_shape=jax.ShapeDtypeStruct((_ET, _D), jnp.float32),
    )(g, w_rel)


@functools.partial(
    pl.kernel,
    mesh=_sc_mesh,
    compiler_params=_sc_params,
    out_type=jax.ShapeDtypeStruct((_NQ, _NP, _CW), jnp.float32),
    scratch_types=[
        pltpu.VMEM_SHARED((_NP, _CW), jnp.float32),
        pltpu.VMEM((_KB, _CHUNK), jnp.int32),
        pltpu.VMEM((_KB, _CHUNK, _CW), jnp.float32),
        pltpu.VMEM((_NPT, _CW), jnp.float32),
        pltpu.SemaphoreType.DMA,
        pltpu.SemaphoreType.DMA,
    ],
)
def _scatter_add(p_hbm, dst_hbm, z_hbm, out_hbm, acc_sh, idx_v, pay_v, buf_v,
                 psem, asem):
    c = lax.axis_index("c")
    s = lax.axis_index("s")
    ept = _ET // 16
    nsup = ept // (_KB * _CHUNK)
    nbase = s * _NPT

    for r in range(_NQ // 2):
        q = 2 * r + c
        pltpu.sync_copy(z_hbm, buf_v)
        pltpu.sync_copy(buf_v, acc_sh.at[pl.ds(nbase, _NPT), :])
        plsc.subcore_barrier()

        def body(si, _, q=q):
            off0 = s * ept + si * (_KB * _CHUNK)
            phs = []
            for j in range(_KB):
                off = off0 + j * _CHUNK
                pltpu.sync_copy(dst_hbm.at[pl.ds(off, _CHUNK)], idx_v.at[j])
                phs.append(pltpu.async_copy(
                    p_hbm.at[pl.ds(off, _CHUNK), pl.ds(q * _CW, _CW)],
                    pay_v.at[j], psem))
            for h in phs:
                h.wait()
            ahs = [pltpu.async_copy(pay_v.at[j], acc_sh.at[idx_v.at[j]], asem,
                                    add=True)
                   for j in range(_KB)]
            for h in ahs:
                h.wait()
            return ()

        lax.fori_loop(0, nsup, body, ())
        plsc.subcore_barrier()
        pltpu.sync_copy(acc_sh.at[pl.ds(nbase, _NPT), :], buf_v)
        pltpu.sync_copy(buf_v, out_hbm.at[q, pl.ds(nbase, _NPT), :])
        plsc.subcore_barrier()


def _post_body(f_ref, a_ref, cw_ref, c2w_ref, g1_ref, b1_ref, g2_ref, b2_ref,
               o_ref):
    acc = jnp.concatenate([a_ref[j] for j in range(_NQ)], axis=-1)
    temp = _mm(f_ref[...], cw_ref[...]) + acc
    u = jnp.maximum(_ln(temp, g1_ref[...], b1_ref[...]), 0.0)
    v = _ln(_mm(u, c2w_ref[...]), g2_ref[...], b2_ref[...])
    o_ref[...] = jnp.maximum(v + f_ref[...], 0.0)


def _post_call(feat, acc4, cw, c2w, g1, b1, g2, b2):
    row = pl.BlockSpec((_TM, _D), lambda i: (i, 0))
    full = pl.BlockSpec((_D, _D), lambda i: (0, 0))
    vec = pl.BlockSpec((1, _D), lambda i: (0, 0))
    return pl.pallas_call(
        _post_body,
        grid=(_NP // _TM,),
        in_specs=[
            row,
            pl.BlockSpec((_NQ, _TM, _CW), lambda i: (0, i, 0)),
            full, full, vec, vec, vec, vec,
        ],
        out_specs=row,
        out_shape=jax.ShapeDtypeStruct((_NP, _D), jnp.float32),
    )(feat, acc4, cw, c2w, g1, b1, g2, b2)


def _prep_indices(pre, suc, left, right):
    srcs, dsts = [], []
    for k2 in range(_S):
        dsts.append(pre[2 * k2])
        srcs.append(pre[2 * k2 + 1])
    for k2 in range(_S):
        dsts.append(suc[2 * k2])
        srcs.append(suc[2 * k2 + 1])
    dsts.append(left[0])
    srcs.append(left[1])
    dsts.append(right[0])
    srcs.append(right[1])
    pad = _EP - _E
    src_all = jnp.concatenate([jnp.pad(x, (0, pad)) for x in srcs])
    dst_all = jnp.concatenate(
        [jnp.pad(x, (0, pad), constant_values=_DUMP) for x in dsts])
    return src_all, dst_all


def kernel(idcs, ctrs, feats, turn, control, intersect, pre, suc, left, right,
           in_w1, in_b1, in_w2, in_gn_g, in_gn_b, seg_w1, seg_b1, seg_w2,
           seg_gn_g, seg_gn_b, ctr_w, pre_w, suc_w, left_w, right_w, norm_g,
           norm_b, ctr2_w, ctr2_gn_g, ctr2_gn_b):
    f32 = jnp.float32
    padn = _NP - _N
    xc = jnp.pad(ctrs.reshape(_N, 2), ((0, padn), (0, _D - 2)))
    xs = jnp.pad(feats[0], ((0, padn), (0, _D - 2)))
    w1p = jnp.pad(in_w1, ((0, _D - 2), (0, 0)))
    sw1p = jnp.pad(seg_w1, ((0, _D - 2), (0, 0)))
    r1 = lambda v: v.reshape(1, _D)

    feat = _mlp_call(xc, xs, w1p, r1(in_b1), in_w2, r1(in_gn_g), r1(in_gn_b),
                     sw1p, r1(seg_b1), seg_w2, r1(seg_gn_g), r1(seg_gn_b))

    src_all, dst_all = _prep_indices(pre, suc, left, right)
    zblk = jnp.zeros((_NPT, _CW), f32)

    for i in range(4):
        w_rel = jnp.concatenate(
            [pre_w[i], suc_w[i], left_w[i][None], right_w[i][None]], axis=0)
        g = _gather_rows(feat, src_all)
        p = _relmm_call(g, w_rel)
        acc4 = _scatter_add(p, dst_all, zblk)
        feat = _post_call(feat, acc4, ctr_w[i], ctr2_w[i], r1(norm_g[i]),
                          r1(norm_b[i]), r1(ctr2_gn_g[i]), r1(ctr2_gn_b[i]))

    return feat[:_N], idcs, ctrs


# speedup vs baseline: 7.0380x; 1.1312x over previous
import functools

import jax
import jax.numpy as jnp
from jax import lax
from jax.experimental import pallas as pl
from jax.experimental.pallas import tpu as pltpu
from jax.experimental.pallas import tpu_sc as plsc

_N = 50000
_D = 128
_S = 6
_E = 30000

_NP = 50176
_EP = 30720
_NREL = 2 * _S + 2
_ET = _NREL * _EP
_NW = 32
_CHUNK = 128
_KB = 7
_KS = 10
_TM = 512
_CW = 16
_NQ = _D // _CW
_NPT = _NP // 16
_DUMP = _N
_EPS = 1e-5


def _ln(x, g, b):
    mu = jnp.mean(x, axis=-1, keepdims=True)
    var = jnp.mean((x - mu) ** 2, axis=-1, keepdims=True)
    return (x - mu) * jax.lax.rsqrt(var + _EPS) * g + b


def _mm(a, b):
    return lax.dot_general(a, b, (((1,), (0,)), ((), ())),
                           preferred_element_type=jnp.float32)


def _mlp_body(xc_ref, xs_ref, w1_ref, b1_ref, w2_ref, gg_ref, gb_ref,
              sw1_ref, sb1_ref, sw2_ref, sgg_ref, sgb_ref, o_ref):
    h1 = jnp.maximum(_mm(xc_ref[...], w1_ref[...]) + b1_ref[...], 0.0)
    h = _ln(_mm(h1, w2_ref[...]), gg_ref[...], gb_ref[...])
    s1 = jnp.maximum(_mm(xs_ref[...], sw1_ref[...]) + sb1_ref[...], 0.0)
    hs = _ln(_mm(s1, sw2_ref[...]), sgg_ref[...], sgb_ref[...])
    o_ref[...] = jnp.maximum(h + hs, 0.0)


def _mlp_call(xc, xs, w1p, b1, w2, gg, gb, sw1p, sb1, sw2, sgg, sgb):
    row = pl.BlockSpec((_TM, _D), lambda i: (i, 0))
    full = pl.BlockSpec((_D, _D), lambda i: (0, 0))
    vec = pl.BlockSpec((1, _D), lambda i: (0, 0))
    return pl.pallas_call(
        _mlp_body,
        grid=(_NP // _TM,),
        in_specs=[row, row, full, vec, full, vec, vec, full, vec, full, vec, vec],
        out_specs=row,
        out_shape=jax.ShapeDtypeStruct((_NP, _D), jnp.float32),
    )(xc, xs, w1p, b1, w2, gg, gb, sw1p, sb1, sw2, sgg, sgb)


_sc_mesh = plsc.VectorSubcoreMesh(core_axis_name="c", subcore_axis_name="s")
_sc_params = pltpu.CompilerParams(use_tc_tiling_on_sc=False)


@functools.partial(
    pl.kernel,
    mesh=_sc_mesh,
    compiler_params=_sc_params,
    out_type=jax.ShapeDtypeStruct((_ET, _D), jnp.float32),
    scratch_types=[
        pltpu.VMEM((_KB, _CHUNK), jnp.int32),
        pltpu.VMEM((_KB, _CHUNK, _D), jnp.float32),
        pltpu.SemaphoreType.DMA,
        pltpu.SemaphoreType.DMA,
    ],
)
def _gather_rows(feat_hbm, src_hbm, out_hbm, idx_v, rows_v, gsem, ssem):
    wid = lax.axis_index("s") * 2 + lax.axis_index("c")
    per_w = _ET // _NW
    base = wid * per_w
    nsup = per_w // (_KB * _CHUNK)

    def body(si, _):
        off0 = base + si * (_KB * _CHUNK)
        pltpu.sync_copy(src_hbm.at[pl.ds(off0 // _CHUNK, _KB), :], idx_v)
        ghs = [pltpu.async_copy(feat_hbm.at[idx_v.at[j]], rows_v.at[j], gsem)
               for j in range(_KB)]
        for h in ghs:
            h.wait()
        shs = [pltpu.async_copy(
                   rows_v.at[j],
                   out_hbm.at[pl.ds(off0 + j * _CHUNK, _CHUNK)], ssem)
               for j in range(_KB)]
        for h in shs:
            h.wait()
        return ()

    lax.fori_loop(0, nsup, body, ())


def _relmm_body(x_ref, w_ref, o_ref):
    o_ref[...] = _mm(x_ref[...], w_ref[0])


def _relmm_call(g, w_rel):
    rows_per_rel = _EP // _TM
    return pl.pallas_call(
        _relmm_body,
        grid=(_ET // _TM,),
        in_specs=[
            pl.BlockSpec((_TM, _D), lambda i: (i, 0)),
            pl.BlockSpec((1, _D, _D), lambda i, r=rows_per_rel: (i // r, 0, 0)),
        ],
        out_specs=pl.BlockSpec((_TM, _D), lambda i: (i, 0)),
        out_shape=jax.ShapeDtypeStruct((_ET, _D), jnp.float32),
    )(g, w_rel)


@functools.partial(
    pl.kernel,
    mesh=_sc_mesh,
    compiler_params=_sc_params,
    out_type=jax.ShapeDtypeStruct((_NQ, _NP, _CW), jnp.float32),
    scratch_types=[
        pltpu.VMEM_SHARED((_NP, _CW), jnp.float32),
        pltpu.VMEM((_KS, _CHUNK), jnp.int32),
        pltpu.VMEM((_KS * _CHUNK, _CW), jnp.float32),
        pltpu.VMEM((_NPT, _CW), jnp.float32),
        pltpu.SemaphoreType.DMA,
        pltpu.SemaphoreType.DMA,
    ],
)
def _scatter_add(p_hbm, dst_hbm, z_hbm, out_hbm, acc_sh, idx_v, pay_v, buf_v,
                 psem, asem):
    c = lax.axis_index("c")
    s = lax.axis_index("s")
    ept = _ET // 16
    nsup = ept // (_KS * _CHUNK)
    nbase = s * _NPT

    for r in range(_NQ // 2):
        q = 2 * r + c
        pltpu.sync_copy(z_hbm, buf_v)
        pltpu.sync_copy(buf_v, acc_sh.at[pl.ds(nbase, _NPT), :])
        plsc.subcore_barrier()

        def body(si, _, q=q):
            off0 = s * ept + si * (_KS * _CHUNK)
            pltpu.sync_copy(dst_hbm.at[pl.ds(off0 // _CHUNK, _KS), :], idx_v)
            pltpu.async_copy(
                p_hbm.at[pl.ds(off0, _KS * _CHUNK), pl.ds(q * _CW, _CW)],
                pay_v, psem).wait()
            ahs = [pltpu.async_copy(
                       pay_v.at[pl.ds(j * _CHUNK, _CHUNK), :],
                       acc_sh.at[idx_v.at[j]], asem, add=True)
                   for j in range(_KS)]
            for h in ahs:
                h.wait()
            return ()

        lax.fori_loop(0, nsup, body, ())
        plsc.subcore_barrier()
        pltpu.sync_copy(acc_sh.at[pl.ds(nbase, _NPT), :], buf_v)
        pltpu.sync_copy(buf_v, out_hbm.at[q, pl.ds(nbase, _NPT), :])
        plsc.subcore_barrier()


def _post_body(f_ref, a_ref, cw_ref, c2w_ref, g1_ref, b1_ref, g2_ref, b2_ref,
               o_ref):
    acc = jnp.concatenate([a_ref[j] for j in range(_NQ)], axis=-1)
    temp = _mm(f_ref[...], cw_ref[...]) + acc
    u = jnp.maximum(_ln(temp, g1_ref[...], b1_ref[...]), 0.0)
    v = _ln(_mm(u, c2w_ref[...]), g2_ref[...], b2_ref[...])
    o_ref[...] = jnp.maximum(v + f_ref[...], 0.0)


def _post_call(feat, acc4, cw, c2w, g1, b1, g2, b2):
    row = pl.BlockSpec((_TM, _D), lambda i: (i, 0))
    full = pl.BlockSpec((_D, _D), lambda i: (0, 0))
    vec = pl.BlockSpec((1, _D), lambda i: (0, 0))
    return pl.pallas_call(
        _post_body,
        grid=(_NP // _TM,),
        in_specs=[
            row,
            pl.BlockSpec((_NQ, _TM, _CW), lambda i: (0, i, 0)),
            full, full, vec, vec, vec, vec,
        ],
        out_specs=row,
        out_shape=jax.ShapeDtypeStruct((_NP, _D), jnp.float32),
    )(feat, acc4, cw, c2w, g1, b1, g2, b2)


def _prep_indices(pre, suc, left, right):
    srcs, dsts = [], []
    for k2 in range(_S):
        dsts.append(pre[2 * k2])
        srcs.append(pre[2 * k2 + 1])
    for k2 in range(_S):
        dsts.append(suc[2 * k2])
        srcs.append(suc[2 * k2 + 1])
    dsts.append(left[0])
    srcs.append(left[1])
    dsts.append(right[0])
    srcs.append(right[1])
    pad = _EP - _E
    src_all = jnp.concatenate([jnp.pad(x, (0, pad)) for x in srcs])
    dst_all = jnp.concatenate(
        [jnp.pad(x, (0, pad), constant_values=_DUMP) for x in dsts])
    return (src_all.reshape(_ET // _CHUNK, _CHUNK),
            dst_all.reshape(_ET // _CHUNK, _CHUNK))


def kernel(idcs, ctrs, feats, turn, control, intersect, pre, suc, left, right,
           in_w1, in_b1, in_w2, in_gn_g, in_gn_b, seg_w1, seg_b1, seg_w2,
           seg_gn_g, seg_gn_b, ctr_w, pre_w, suc_w, left_w, right_w, norm_g,
           norm_b, ctr2_w, ctr2_gn_g, ctr2_gn_b):
    f32 = jnp.float32
    padn = _NP - _N
    xc = jnp.pad(ctrs.reshape(_N, 2), ((0, padn), (0, _D - 2)))
    xs = jnp.pad(feats[0], ((0, padn), (0, _D - 2)))
    w1p = jnp.pad(in_w1, ((0, _D - 2), (0, 0)))
    sw1p = jnp.pad(seg_w1, ((0, _D - 2), (0, 0)))
    r1 = lambda v: v.reshape(1, _D)

    feat = _mlp_call(xc, xs, w1p, r1(in_b1), in_w2, r1(in_gn_g), r1(in_gn_b),
                     sw1p, r1(seg_b1), seg_w2, r1(seg_gn_g), r1(seg_gn_b))

    src_all, dst_all = _prep_indices(pre, suc, left, right)
    zblk = jnp.zeros((_NPT, _CW), f32)

    for i in range(4):
        w_rel = jnp.concatenate(
            [pre_w[i], suc_w[i], left_w[i][None], right_w[i][None]], axis=0)
        g = _gather_rows(feat, src_all)
        p = _relmm_call(g, w_rel)
        acc4 = _scatter_add(p, dst_all, zblk)
        feat = _post_call(feat, acc4, ctr_w[i], ctr2_w[i], r1(norm_g[i]),
                          r1(norm_b[i]), r1(ctr2_gn_g[i]), r1(ctr2_gn_b[i]))

    return feat[:_N], idcs, ctrs


# speedup vs baseline: 7.5987x; 1.0797x over previous
import functools

import jax
import jax.numpy as jnp
from jax import lax
from jax.experimental import pallas as pl
from jax.experimental.pallas import tpu as pltpu
from jax.experimental.pallas import tpu_sc as plsc

_N = 50000
_D = 128
_S = 6
_E = 30000

_NP = 50176
_EP = 30720
_NREL = 2 * _S + 2
_ET = _NREL * _EP
_NW = 32
_CHUNK = 128
_KB = 7
_KS = 10
_TM = 512
_CW = 16
_NQ = _D // _CW
_NPT = _NP // 16
_DUMP = _N
_EPS = 1e-5


def _ln(x, g, b):
    mu = jnp.mean(x, axis=-1, keepdims=True)
    var = jnp.mean((x - mu) ** 2, axis=-1, keepdims=True)
    return (x - mu) * jax.lax.rsqrt(var + _EPS) * g + b


def _mm(a, b):
    return lax.dot_general(a, b, (((1,), (0,)), ((), ())),
                           preferred_element_type=jnp.float32)


def _mlp_body(xc_ref, xs_ref, w1_ref, b1_ref, w2_ref, gg_ref, gb_ref,
              sw1_ref, sb1_ref, sw2_ref, sgg_ref, sgb_ref, o_ref):
    h1 = jnp.maximum(_mm(xc_ref[...], w1_ref[...]) + b1_ref[...], 0.0)
    h = _ln(_mm(h1, w2_ref[...]), gg_ref[...], gb_ref[...])
    s1 = jnp.maximum(_mm(xs_ref[...], sw1_ref[...]) + sb1_ref[...], 0.0)
    hs = _ln(_mm(s1, sw2_ref[...]), sgg_ref[...], sgb_ref[...])
    o_ref[...] = jnp.maximum(h + hs, 0.0)


def _mlp_call(xc, xs, w1p, b1, w2, gg, gb, sw1p, sb1, sw2, sgg, sgb):
    row = pl.BlockSpec((_TM, _D), lambda i: (i, 0))
    full = pl.BlockSpec((_D, _D), lambda i: (0, 0))
    vec = pl.BlockSpec((1, _D), lambda i: (0, 0))
    return pl.pallas_call(
        _mlp_body,
        grid=(_NP // _TM,),
        in_specs=[row, row, full, vec, full, vec, vec, full, vec, full, vec, vec],
        out_specs=row,
        out_shape=jax.ShapeDtypeStruct((_NP, _D), jnp.float32),
    )(xc, xs, w1p, b1, w2, gg, gb, sw1p, sb1, sw2, sgg, sgb)


_sc_mesh = plsc.VectorSubcoreMesh(core_axis_name="c", subcore_axis_name="s")
_sc_params = pltpu.CompilerParams(use_tc_tiling_on_sc=False)


@functools.partial(
    pl.kernel,
    mesh=_sc_mesh,
    compiler_params=_sc_params,
    out_type=jax.ShapeDtypeStruct((_ET, _D), jnp.float32),
    scratch_types=[
        pltpu.VMEM((_KB, _CHUNK), jnp.int32),
        pltpu.VMEM((_KB, _CHUNK, _D), jnp.float32),
        pltpu.SemaphoreType.DMA,
        pltpu.SemaphoreType.DMA,
    ],
)
def _gather_rows(feat_hbm, src_hbm, out_hbm, idx_v, rows_v, gsem, ssem):
    wid = lax.axis_index("s") * 2 + lax.axis_index("c")
    per_w = _ET // _NW
    base = wid * per_w
    nsup = per_w // (_KB * _CHUNK)

    def body(si, _):
        off0 = base + si * (_KB * _CHUNK)
        pltpu.sync_copy(src_hbm.at[pl.ds(off0 // _CHUNK, _KB), :], idx_v)
        ghs = [pltpu.async_copy(feat_hbm.at[idx_v.at[j]], rows_v.at[j], gsem)
               for j in range(_KB)]
        for h in ghs:
            h.wait()
        shs = [pltpu.async_copy(
                   rows_v.at[j],
                   out_hbm.at[pl.ds(off0 + j * _CHUNK, _CHUNK)], ssem)
               for j in range(_KB)]
        for h in shs:
            h.wait()
        return ()

    lax.fori_loop(0, nsup, body, ())


def _relmm_body(x_ref, w_ref, o_ref):
    o_ref[...] = _mm(x_ref[...], w_ref[0])


def _relmm_call(g, w_rel):
    rows_per_rel = _EP // _TM
    return pl.pallas_call(
        _relmm_body,
        grid=(_ET // _TM,),
        in_specs=[
            pl.BlockSpec((_TM, _D), lambda i: (i, 0)),
            pl.BlockSpec((1, _D, _D), lambda i, r=rows_per_rel: (i // r, 0, 0)),
        ],
        out_specs=pl.BlockSpec((_TM, _D), lambda i: (i, 0)),
        out_shape=jax.ShapeDtypeStruct((_ET, _D), jnp.float32),
    )(g, w_rel)


@functools.partial(
    pl.kernel,
    mesh=_sc_mesh,
    compiler_params=_sc_params,
    out_type=jax.ShapeDtypeStruct((_NQ, _NP, _CW), jnp.float32),
    scratch_types=[
        pltpu.VMEM_SHARED((_NP, _CW), jnp.float32),
        pltpu.VMEM((2, _KS, _CHUNK), jnp.int32),
        pltpu.VMEM((2, _KS * _CHUNK, _CW), jnp.float32),
        pltpu.VMEM((_NPT // 2, _CW), jnp.float32),
        pltpu.SemaphoreType.DMA,
        pltpu.SemaphoreType.DMA,
        pltpu.SemaphoreType.DMA,
        pltpu.SemaphoreType.DMA,
        pltpu.SemaphoreType.DMA,
    ],
)
def _scatter_add(p_hbm, dst_hbm, z_hbm, out_hbm, acc_sh, idx_v, pay_v, buf_v,
                 isem0, isem1, psem0, psem1, asem):
    c = lax.axis_index("c")
    s = lax.axis_index("s")
    ept = _ET // 16
    nsup = ept // (_KS * _CHUNK)
    nbase = s * _NPT
    isems = (isem0, isem1)
    psems = (psem0, psem1)
    half = _NPT // 2

    def fire(su, b, q):
        off0 = s * ept + su * (_KS * _CHUNK)
        pltpu.async_copy(dst_hbm.at[pl.ds(off0 // _CHUNK, _KS), :],
                         idx_v.at[b], isems[b])
        pltpu.async_copy(
            p_hbm.at[pl.ds(off0, _KS * _CHUNK), pl.ds(q * _CW, _CW)],
            pay_v.at[b], psems[b])

    def drain_and_add(b):
        pltpu.make_async_copy(dst_hbm.at[pl.ds(0, _KS), :], idx_v.at[b],
                              isems[b]).wait()
        pltpu.make_async_copy(p_hbm.at[pl.ds(0, _KS * _CHUNK),
                                       pl.ds(0, _CW)],
                              pay_v.at[b], psems[b]).wait()
        ahs = [pltpu.async_copy(
                   pay_v.at[b].at[pl.ds(j * _CHUNK, _CHUNK), :],
                   acc_sh.at[idx_v.at[b].at[j]], asem, add=True)
               for j in range(_KS)]
        for h in ahs:
            h.wait()

    for r in range(_NQ // 2):
        q = 2 * r + c
        for hlf in range(2):
            pltpu.sync_copy(z_hbm.at[pl.ds(hlf * half, half), :], buf_v)
            pltpu.sync_copy(buf_v,
                            acc_sh.at[pl.ds(nbase + hlf * half, half), :])
        plsc.subcore_barrier()

        fire(0, 0, q)

        def body(k, _, q=q):
            fire(2 * k + 1, 1, q)
            drain_and_add(0)
            fire(2 * k + 2, 0, q)
            drain_and_add(1)
            return ()

        lax.fori_loop(0, (nsup - 1) // 2, body, ())
        drain_and_add(0)
        plsc.subcore_barrier()
        for hlf in range(2):
            pltpu.sync_copy(acc_sh.at[pl.ds(nbase + hlf * half, half), :],
                            buf_v)
            pltpu.sync_copy(buf_v,
                            out_hbm.at[q, pl.ds(nbase + hlf * half, half), :])
        plsc.subcore_barrier()


def _post_body(f_ref, a_ref, cw_ref, c2w_ref, g1_ref, b1_ref, g2_ref, b2_ref,
               o_ref):
    acc = jnp.concatenate([a_ref[j] for j in range(_NQ)], axis=-1)
    temp = _mm(f_ref[...], cw_ref[...]) + acc
    u = jnp.maximum(_ln(temp, g1_ref[...], b1_ref[...]), 0.0)
    v = _ln(_mm(u, c2w_ref[...]), g2_ref[...], b2_ref[...])
    o_ref[...] = jnp.maximum(v + f_ref[...], 0.0)


def _post_call(feat, acc4, cw, c2w, g1, b1, g2, b2):
    row = pl.BlockSpec((_TM, _D), lambda i: (i, 0))
    full = pl.BlockSpec((_D, _D), lambda i: (0, 0))
    vec = pl.BlockSpec((1, _D), lambda i: (0, 0))
    return pl.pallas_call(
        _post_body,
        grid=(_NP // _TM,),
        in_specs=[
            row,
            pl.BlockSpec((_NQ, _TM, _CW), lambda i: (0, i, 0)),
            full, full, vec, vec, vec, vec,
        ],
        out_specs=row,
        out_shape=jax.ShapeDtypeStruct((_NP, _D), jnp.float32),
    )(feat, acc4, cw, c2w, g1, b1, g2, b2)


def _prep_indices(pre, suc, left, right):
    srcs, dsts = [], []
    for k2 in range(_S):
        dsts.append(pre[2 * k2])
        srcs.append(pre[2 * k2 + 1])
    for k2 in range(_S):
        dsts.append(suc[2 * k2])
        srcs.append(suc[2 * k2 + 1])
    dsts.append(left[0])
    srcs.append(left[1])
    dsts.append(right[0])
    srcs.append(right[1])
    pad = _EP - _E
    src_all = jnp.concatenate([jnp.pad(x, (0, pad)) for x in srcs])
    dst_all = jnp.concatenate(
        [jnp.pad(x, (0, pad), constant_values=_DUMP) for x in dsts])
    return (src_all.reshape(_ET // _CHUNK, _CHUNK),
            dst_all.reshape(_ET // _CHUNK, _CHUNK))


def kernel(idcs, ctrs, feats, turn, control, intersect, pre, suc, left, right,
           in_w1, in_b1, in_w2, in_gn_g, in_gn_b, seg_w1, seg_b1, seg_w2,
           seg_gn_g, seg_gn_b, ctr_w, pre_w, suc_w, left_w, right_w, norm_g,
           norm_b, ctr2_w, ctr2_gn_g, ctr2_gn_b):
    f32 = jnp.float32
    padn = _NP - _N
    xc = jnp.pad(ctrs.reshape(_N, 2), ((0, padn), (0, _D - 2)))
    xs = jnp.pad(feats[0], ((0, padn), (0, _D - 2)))
    w1p = jnp.pad(in_w1, ((0, _D - 2), (0, 0)))
    sw1p = jnp.pad(seg_w1, ((0, _D - 2), (0, 0)))
    r1 = lambda v: v.reshape(1, _D)

    feat = _mlp_call(xc, xs, w1p, r1(in_b1), in_w2, r1(in_gn_g), r1(in_gn_b),
                     sw1p, r1(seg_b1), seg_w2, r1(seg_gn_g), r1(seg_gn_b))

    src_all, dst_all = _prep_indices(pre, suc, left, right)
    zblk = jnp.zeros((_NPT, _CW), f32)

    for i in range(4):
        w_rel = jnp.concatenate(
            [pre_w[i], suc_w[i], left_w[i][None], right_w[i][None]], axis=0)
        g = _gather_rows(feat, src_all)
        p = _relmm_call(g, w_rel)
        acc4 = _scatter_add(p, dst_all, zblk)
        feat = _post_call(feat, acc4, ctr_w[i], ctr2_w[i], r1(norm_g[i]),
                          r1(norm_b[i]), r1(ctr2_gn_g[i]), r1(ctr2_gn_b[i]))

    return feat[:_N], idcs, ctrs
